# Initial kernel scaffold; baseline (speedup 1.0000x reference)
import jax
import jax.numpy as jnp
from jax.experimental import pallas as pl


def kernel(inputs):
    raise NotImplementedError("write your pallas kernel here")


# speedup vs baseline: 16.4845x; 16.4845x over previous
import jax
import jax.numpy as jnp
from jax.experimental import pallas as pl

TOPK = 8


def _topk_block(in_ref, out_ref):
    x = in_ref[0]
    s, cb = x.shape
    row = jax.lax.broadcasted_iota(jnp.int32, (s, cb), 0)
    neg_inf = jnp.float32(-jnp.inf)
    for k in range(TOPK):
        m = jnp.max(x, axis=0, keepdims=True)
        out_ref[0, k, :] = m[0]
        hit = x == m
        first = jnp.min(jnp.where(hit, row, s), axis=0, keepdims=True)
        x = jnp.where(row == first, neg_inf, x)


def kernel(inputs):
    b, s, c = inputs.shape
    cb = 128
    out = pl.pallas_call(
        _topk_block,
        grid=(b, c // cb),
        in_specs=[pl.BlockSpec((1, s, cb), lambda i, j: (i, 0, j))],
        out_specs=pl.BlockSpec((1, TOPK, cb), lambda i, j: (i, 0, j)),
        out_shape=jax.ShapeDtypeStruct((b, TOPK, c), jnp.float32),
    )(inputs)
    return out.transpose(0, 2, 1).reshape(b, c * TOPK)


# speedup vs baseline: 33.2688x; 2.0182x over previous
import functools

import jax
import jax.numpy as jnp
from jax import lax
from jax.experimental import pallas as pl
from jax.experimental.pallas import tpu as pltpu
from jax.experimental.pallas import tpu_sc as plsc

TOPK = 8
LANES = 16
NWORKERS = 32

_SORT8 = ((0, 2), (1, 3), (4, 6), (5, 7),
          (0, 4), (1, 5), (2, 6), (3, 7),
          (0, 1), (2, 3), (4, 5), (6, 7),
          (2, 4), (3, 5), (1, 4), (3, 6),
          (1, 2), (3, 4), (5, 6))
_BITONIC8 = ((0, 4), (1, 5), (2, 6), (3, 7),
             (0, 2), (1, 3), (4, 6), (5, 7),
             (0, 1), (2, 3), (4, 5), (6, 7))


def _sort_net(vals, net):
    for i, j in net:
        a, b = vals[i], vals[j]
        vals[i] = jnp.minimum(a, b)
        vals[j] = jnp.maximum(a, b)
    return vals


def _merge_top8(top, other):
    merged = [jnp.maximum(top[m], other[7 - m]) for m in range(8)]
    return _sort_net(merged, _BITONIC8)


def kernel(inputs):
    b_dim, s_dim, c_dim = inputs.shape
    ncb = c_dim // 128
    nhalf = 2
    half_rows = s_dim // nhalf
    chunk_rows = 256
    nchunks = half_rows // chunk_rows
    ntasks1 = b_dim * ncb * nhalf
    tper1 = ntasks1 // NWORKERS
    ntasks2 = b_dim * ncb
    mesh = plsc.VectorSubcoreMesh(core_axis_name="c", subcore_axis_name="s")
    neg_inf = jnp.float32(-jnp.inf)

    @functools.partial(
        pl.kernel,
        mesh=mesh,
        out_type=jax.ShapeDtypeStruct((b_dim, ncb, nhalf, TOPK, 128),
                                      jnp.float32),
        scratch_types=[
            pltpu.VMEM((chunk_rows, 128), jnp.float32),
            pltpu.VMEM((8, TOPK, LANES), jnp.float32),
            pltpu.VMEM((TOPK, 128), jnp.float32),
        ],
    )
    def phase1(in_hbm, cand_hbm, slab, tbuf, outbuf):
        wid = lax.axis_index("s") * 2 + lax.axis_index("c")

        def task_body(j, _):
            t = wid * tper1 + j
            b = t // (ncb * nhalf)
            r = t % (ncb * nhalf)
            cb = r // nhalf
            h = r % nhalf

            def init_g(g, _):
                for k in range(TOPK):
                    tbuf[g, k] = jnp.full((LANES,), neg_inf, jnp.float32)
                return 0

            lax.fori_loop(0, 8, init_g, 0)

            def chunk_body(ci, _):
                row0 = h * half_rows + ci * chunk_rows
                pltpu.sync_copy(
                    in_hbm.at[b, pl.ds(row0, chunk_rows),
                              pl.ds(cb * 128, 128)], slab)

                def group_body(g, _):
                    top = tuple(tbuf[g, k] for k in range(TOPK))

                    def seg_body(i, carry):
                        seg = [slab[i * 8 + k, pl.ds(g * LANES, LANES)]
                               for k in range(8)]
                        seg = _sort_net(seg, _SORT8)
                        return tuple(_merge_top8(list(carry), seg))

                    top = lax.fori_loop(0, chunk_rows // 8, seg_body, top)
                    for k in range(TOPK):
                        tbuf[g, k] = top[k]
                    return 0

                lax.fori_loop(0, 8, group_body, 0)
                return 0

            lax.fori_loop(0, nchunks, chunk_body, 0)

            def out_g(g, _):
                for k in range(TOPK):
                    outbuf[k, pl.ds(g * LANES, LANES)] = tbuf[g, k]
                return 0

            lax.fori_loop(0, 8, out_g, 0)
            pltpu.sync_copy(outbuf, cand_hbm.at[b, cb, h])
            return 0

        lax.fori_loop(0, tper1, task_body, 0)

    @functools.partial(
        pl.kernel,
        mesh=mesh,
        out_type=jax.ShapeDtypeStruct((b_dim, TOPK, c_dim), jnp.float32),
        scratch_types=[
            pltpu.VMEM((nhalf, TOPK, 128), jnp.float32),
            pltpu.VMEM((TOPK, 128), jnp.float32),
        ],
    )
    def phase2(cand_hbm, out_hbm, slab, outbuf):
        wid = lax.axis_index("s") * 2 + lax.axis_index("c")

        def task_body(t, _):
            b = t // ncb
            cb = t % ncb
            pltpu.sync_copy(cand_hbm.at[b, cb], slab)

            def group_body(g, _):
                a = [slab[0, k, pl.ds(g * LANES, LANES)] for k in range(TOPK)]
                c = [slab[1, k, pl.ds(g * LANES, LANES)] for k in range(TOPK)]
                top = _merge_top8(a, c)
                for k in range(TOPK):
                    outbuf[k, pl.ds(g * LANES, LANES)] = top[7 - k]
                return 0

            lax.fori_loop(0, 8, group_body, 0)
            pltpu.sync_copy(outbuf, out_hbm.at[b, :, pl.ds(cb * 128, 128)])
            return 0

        ntask_here = jnp.where(wid < ntasks2 - NWORKERS, 2, 1)

        def strided_loop(i, _):
            return task_body(wid + i * NWORKERS, 0)

        lax.fori_loop(0, ntask_here, strided_loop, 0)

    cand = phase1(inputs)
    out_kc = phase2(cand)
    return out_kc.transpose(0, 2, 1).reshape(b_dim, c_dim * TOPK)


# speedup vs baseline: 45.2932x; 1.3614x over previous
import functools

import jax
import jax.numpy as jnp
from jax import lax
from jax.experimental import pallas as pl
from jax.experimental.pallas import tpu as pltpu
from jax.experimental.pallas import tpu_sc as plsc

TOPK = 8
LANES = 16
NWORKERS = 32

_SORT8 = ((0, 2), (1, 3), (4, 6), (5, 7),
          (0, 4), (1, 5), (2, 6), (3, 7),
          (0, 1), (2, 3), (4, 5), (6, 7),
          (2, 4), (3, 5), (1, 4), (3, 6),
          (1, 2), (3, 4), (5, 6))
_BITONIC8 = ((0, 4), (1, 5), (2, 6), (3, 7),
             (0, 2), (1, 3), (4, 6), (5, 7),
             (0, 1), (2, 3), (4, 5), (6, 7))


def _sort_net(vals, net):
    for i, j in net:
        a, b = vals[i], vals[j]
        vals[i] = jnp.minimum(a, b)
        vals[j] = jnp.maximum(a, b)
    return vals


def _merge_top8(top, other):
    merged = [jnp.maximum(top[m], other[7 - m]) for m in range(8)]
    return _sort_net(merged, _BITONIC8)


def kernel(inputs):
    b_dim, s_dim, c_dim = inputs.shape
    ncb = c_dim // 128
    nhalf = 2
    half_rows = s_dim // nhalf
    chunk_rows = 256
    nchunks = half_rows // chunk_rows
    ntasks1 = b_dim * ncb * nhalf
    tper1 = ntasks1 // NWORKERS
    ntasks2 = b_dim * ncb
    mesh = plsc.VectorSubcoreMesh(core_axis_name="c", subcore_axis_name="s")
    neg_inf = jnp.float32(-jnp.inf)

    @functools.partial(
        pl.kernel,
        mesh=mesh,
        out_type=jax.ShapeDtypeStruct((b_dim, ncb, nhalf, TOPK, 128),
                                      jnp.float32),
        scratch_types=[
            pltpu.VMEM((chunk_rows, 128), jnp.float32),
            pltpu.VMEM((chunk_rows, 128), jnp.float32),
            pltpu.VMEM((8, TOPK, LANES), jnp.float32),
            pltpu.VMEM((TOPK, 128), jnp.float32),
            pltpu.SemaphoreType.DMA,
            pltpu.SemaphoreType.DMA,
        ],
    )
    def phase1(in_hbm, cand_hbm, slab0, slab1, tbuf, outbuf, sem0, sem1):
        wid = lax.axis_index("s") * 2 + lax.axis_index("c")
        slabs = (slab0, slab1)
        sems = (sem0, sem1)

        def chunk_src(t, ci):
            b = t // (ncb * nhalf)
            r = t % (ncb * nhalf)
            cb = r // nhalf
            h = r % nhalf
            row0 = h * half_rows + ci * chunk_rows
            return in_hbm.at[b, pl.ds(row0, chunk_rows), pl.ds(cb * 128, 128)]

        def start_fetch(t, ci, ring):
            pltpu.async_copy(chunk_src(t, ci), slabs[ring], sems[ring])

        def compute_chunk(slab, tbuf):
            def group_body(g, _):
                top = tuple(tbuf[g, k] for k in range(TOPK))

                def seg_body(i, carry):
                    seg = [slab[i * 8 + k, pl.ds(g * LANES, LANES)]
                           for k in range(8)]
                    seg = _sort_net(seg, _SORT8)
                    return tuple(_merge_top8(list(carry), seg))

                top = lax.fori_loop(0, chunk_rows // 8, seg_body, top)
                for k in range(TOPK):
                    tbuf[g, k] = top[k]
                return 0

            lax.fori_loop(0, 8, group_body, 0)

        start_fetch(wid * tper1, 0, 0)

        def task_body(j, _):
            t = wid * tper1 + j

            def init_g(g, _):
                for k in range(TOPK):
                    tbuf[g, k] = jnp.full((LANES,), neg_inf, jnp.float32)
                return 0

            lax.fori_loop(0, 8, init_g, 0)

            def pair_body(p, _):
                for sub in range(2):
                    ci = 2 * p + sub
                    ring = sub
                    nci = ci + 1

                    @pl.when(jnp.logical_or(nci < nchunks, j < tper1 - 1))
                    def _():
                        nt = jnp.where(nci < nchunks, t, t + 1)
                        pltpu.async_copy(
                            chunk_src(nt, lax.rem(nci, nchunks)),
                            slabs[(ring + 1) % 2], sems[(ring + 1) % 2])

                    pltpu.make_async_copy(
                        chunk_src(t, ci), slabs[ring], sems[ring]).wait()
                    compute_chunk(slabs[ring], tbuf)
                return 0

            lax.fori_loop(0, nchunks // 2, pair_body, 0)

            b = t // (ncb * nhalf)
            r = t % (ncb * nhalf)
            cb = r // nhalf
            h = r % nhalf

            def out_g(g, _):
                for k in range(TOPK):
                    outbuf[k, pl.ds(g * LANES, LANES)] = tbuf[g, k]
                return 0

            lax.fori_loop(0, 8, out_g, 0)
            pltpu.sync_copy(outbuf, cand_hbm.at[b, cb, h])
            return 0

        lax.fori_loop(0, tper1, task_body, 0)

    @functools.partial(
        pl.kernel,
        mesh=mesh,
        out_type=jax.ShapeDtypeStruct((b_dim, TOPK, c_dim), jnp.float32),
        scratch_types=[
            pltpu.VMEM((nhalf, TOPK, 128), jnp.float32),
            pltpu.VMEM((TOPK, 128), jnp.float32),
        ],
    )
    def phase2(cand_hbm, out_hbm, slab, outbuf):
        wid = lax.axis_index("s") * 2 + lax.axis_index("c")

        def task_body(t, _):
            b = t // ncb
            cb = t % ncb
            pltpu.sync_copy(cand_hbm.at[b, cb], slab)

            def group_body(g, _):
                a = [slab[0, k, pl.ds(g * LANES, LANES)] for k in range(TOPK)]
                c = [slab[1, k, pl.ds(g * LANES, LANES)] for k in range(TOPK)]
                top = _merge_top8(a, c)
                for k in range(TOPK):
                    outbuf[k, pl.ds(g * LANES, LANES)] = top[7 - k]
                return 0

            lax.fori_loop(0, 8, group_body, 0)
            pltpu.sync_copy(outbuf, out_hbm.at[b, :, pl.ds(cb * 128, 128)])
            return 0

        ntask_here = jnp.where(wid < ntasks2 - NWORKERS, 2, 1)

        def strided_loop(i, _):
            return task_body(wid + i * NWORKERS, 0)

        lax.fori_loop(0, ntask_here, strided_loop, 0)

    cand = phase1(inputs)
    out_kc = phase2(cand)
    return out_kc.transpose(0, 2, 1).reshape(b_dim, c_dim * TOPK)


# speedup vs baseline: 45.6749x; 1.0084x over previous
import functools

import jax
import jax.numpy as jnp
from jax import lax
from jax.experimental import pallas as pl
from jax.experimental.pallas import tpu as pltpu
from jax.experimental.pallas import tpu_sc as plsc

TOPK = 8
LANES = 16
NWORKERS = 32

_SORT8 = ((0, 2), (1, 3), (4, 6), (5, 7),
          (0, 4), (1, 5), (2, 6), (3, 7),
          (0, 1), (2, 3), (4, 5), (6, 7),
          (2, 4), (3, 5), (1, 4), (3, 6),
          (1, 2), (3, 4), (5, 6))
_BITONIC8 = ((0, 4), (1, 5), (2, 6), (3, 7),
             (0, 2), (1, 3), (4, 6), (5, 7),
             (0, 1), (2, 3), (4, 5), (6, 7))


def _sort_net(vals, net):
    for i, j in net:
        a, b = vals[i], vals[j]
        vals[i] = jnp.minimum(a, b)
        vals[j] = jnp.maximum(a, b)
    return vals


def _merge_top8(top, other):
    merged = [jnp.maximum(top[m], other[7 - m]) for m in range(8)]
    return _sort_net(merged, _BITONIC8)


def kernel(inputs):
    b_dim, s_dim, c_dim = inputs.shape
    ncb = c_dim // 128
    nhalf = 2
    half_rows = s_dim // nhalf
    chunk_rows = 256
    nchunks = half_rows // chunk_rows
    ntasks1 = b_dim * ncb * nhalf
    tper1 = ntasks1 // NWORKERS
    ntasks2 = b_dim * ncb
    mesh = plsc.VectorSubcoreMesh(core_axis_name="c", subcore_axis_name="s")
    neg_inf = jnp.float32(-jnp.inf)

    @functools.partial(
        pl.kernel,
        mesh=mesh,
        out_type=jax.ShapeDtypeStruct((b_dim, ncb, nhalf, TOPK, 128),
                                      jnp.float32),
        scratch_types=[
            pltpu.VMEM((chunk_rows, 128), jnp.float32),
            pltpu.VMEM((chunk_rows, 128), jnp.float32),
            pltpu.VMEM((8, TOPK, LANES), jnp.float32),
            pltpu.VMEM((TOPK, 128), jnp.float32),
            pltpu.SemaphoreType.DMA,
            pltpu.SemaphoreType.DMA,
        ],
    )
    def phase1(in_hbm, cand_hbm, slab0, slab1, tbuf, outbuf, sem0, sem1):
        wid = lax.axis_index("s") * 2 + lax.axis_index("c")
        slabs = (slab0, slab1)
        sems = (sem0, sem1)

        def chunk_src(t, ci):
            b = t // (ncb * nhalf)
            r = t % (ncb * nhalf)
            cb = r // nhalf
            h = r % nhalf
            row0 = h * half_rows + ci * chunk_rows
            return in_hbm.at[b, pl.ds(row0, chunk_rows), pl.ds(cb * 128, 128)]

        def start_fetch(t, ci, ring):
            pltpu.async_copy(chunk_src(t, ci), slabs[ring], sems[ring])

        def compute_chunk(slab, tbuf):
            def group_body(g, _):
                top = tuple(tbuf[g, k] for k in range(TOPK))

                def seg_body(i, carry):
                    rows = [slab[i * 16 + k, pl.ds(g * LANES, LANES)]
                            for k in range(16)]
                    sa = _sort_net(rows[:8], _SORT8)
                    sb = _sort_net(rows[8:], _SORT8)
                    pair = _merge_top8(sa, sb)
                    return tuple(_merge_top8(list(carry), pair))

                top = lax.fori_loop(0, chunk_rows // 16, seg_body, top)
                for k in range(TOPK):
                    tbuf[g, k] = top[k]
                return 0

            lax.fori_loop(0, 8, group_body, 0)

        start_fetch(wid * tper1, 0, 0)

        def task_body(j, _):
            t = wid * tper1 + j

            def init_g(g, _):
                for k in range(TOPK):
                    tbuf[g, k] = jnp.full((LANES,), neg_inf, jnp.float32)
                return 0

            lax.fori_loop(0, 8, init_g, 0)

            def pair_body(p, _):
                for sub in range(2):
                    ci = 2 * p + sub
                    ring = sub
                    nci = ci + 1

                    @pl.when(jnp.logical_or(nci < nchunks, j < tper1 - 1))
                    def _():
                        nt = jnp.where(nci < nchunks, t, t + 1)
                        pltpu.async_copy(
                            chunk_src(nt, lax.rem(nci, nchunks)),
                            slabs[(ring + 1) % 2], sems[(ring + 1) % 2])

                    pltpu.make_async_copy(
                        chunk_src(t, ci), slabs[ring], sems[ring]).wait()
                    compute_chunk(slabs[ring], tbuf)
                return 0

            lax.fori_loop(0, nchunks // 2, pair_body, 0)

            b = t // (ncb * nhalf)
            r = t % (ncb * nhalf)
            cb = r // nhalf
            h = r % nhalf

            def out_g(g, _):
                for k in range(TOPK):
                    outbuf[k, pl.ds(g * LANES, LANES)] = tbuf[g, k]
                return 0

            lax.fori_loop(0, 8, out_g, 0)
            pltpu.sync_copy(outbuf, cand_hbm.at[b, cb, h])
            return 0

        lax.fori_loop(0, tper1, task_body, 0)

    @functools.partial(
        pl.kernel,
        mesh=mesh,
        out_type=jax.ShapeDtypeStruct((b_dim, TOPK, c_dim), jnp.float32),
        scratch_types=[
            pltpu.VMEM((nhalf, TOPK, 128), jnp.float32),
            pltpu.VMEM((TOPK, 128), jnp.float32),
        ],
    )
    def phase2(cand_hbm, out_hbm, slab, outbuf):
        wid = lax.axis_index("s") * 2 + lax.axis_index("c")

        def task_body(t, _):
            b = t // ncb
            cb = t % ncb
            pltpu.sync_copy(cand_hbm.at[b, cb], slab)

            def group_body(g, _):
                a = [slab[0, k, pl.ds(g * LANES, LANES)] for k in range(TOPK)]
                c = [slab[1, k, pl.ds(g * LANES, LANES)] for k in range(TOPK)]
                top = _merge_top8(a, c)
                for k in range(TOPK):
                    outbuf[k, pl.ds(g * LANES, LANES)] = top[7 - k]
                return 0

            lax.fori_loop(0, 8, group_body, 0)
            pltpu.sync_copy(outbuf, out_hbm.at[b, :, pl.ds(cb * 128, 128)])
            return 0

        ntask_here = jnp.where(wid < ntasks2 - NWORKERS, 2, 1)

        def strided_loop(i, _):
            return task_body(wid + i * NWORKERS, 0)

        lax.fori_loop(0, ntask_here, strided_loop, 0)

    cand = phase1(inputs)
    out_kc = phase2(cand)
    return out_kc.transpose(0, 2, 1).reshape(b_dim, c_dim * TOPK)


# speedup vs baseline: 49.8221x; 1.0908x over previous
import functools

import jax
import jax.numpy as jnp
from jax import lax
from jax.experimental import pallas as pl
from jax.experimental.pallas import tpu as pltpu
from jax.experimental.pallas import tpu_sc as plsc

TOPK = 8
LANES = 16
NWORKERS = 32
NB_TC = 4

_SORT8 = ((0, 2), (1, 3), (4, 6), (5, 7),
          (0, 4), (1, 5), (2, 6), (3, 7),
          (0, 1), (2, 3), (4, 5), (6, 7),
          (2, 4), (3, 5), (1, 4), (3, 6),
          (1, 2), (3, 4), (5, 6))
_BITONIC8 = ((0, 4), (1, 5), (2, 6), (3, 7),
             (0, 2), (1, 3), (4, 6), (5, 7),
             (0, 1), (2, 3), (4, 5), (6, 7))


def _sort_net(vals, net):
    for i, j in net:
        a, b = vals[i], vals[j]
        vals[i] = jnp.minimum(a, b)
        vals[j] = jnp.maximum(a, b)
    return vals


def _merge_top8(top, other):
    merged = [jnp.maximum(top[m], other[7 - m]) for m in range(8)]
    return _sort_net(merged, _BITONIC8)


def _tc_block(in_ref, out_ref):
    x = in_ref[0]
    s_rows = x.shape[0]
    blk = s_rows // 8
    s = [x[blk * k:blk * (k + 1), :] for k in range(8)]
    s = _sort_net(s, _SORT8)
    length = blk
    while length > 1:
        h = length // 2
        lo = [v[:h] for v in s]
        hi = [v[h:] for v in s]
        m = [jnp.maximum(lo[i], hi[7 - i]) for i in range(8)]
        s = _sort_net(m, _BITONIC8)
        length = h
    for k in range(TOPK):
        out_ref[0, k, :] = s[7 - k][0]


def _tc_topk(inputs, nb):
    b_dim, s_dim, c_dim = inputs.shape
    return pl.pallas_call(
        _tc_block,
        grid=(nb, c_dim // 128),
        in_specs=[pl.BlockSpec((1, s_dim, 128), lambda i, j: (i, 0, j))],
        out_specs=pl.BlockSpec((1, TOPK, 128), lambda i, j: (i, 0, j)),
        out_shape=jax.ShapeDtypeStruct((nb, TOPK, c_dim), jnp.float32),
    )(inputs)


def kernel(inputs):
    b_dim, s_dim, c_dim = inputs.shape
    ncb = c_dim // 128
    nb_sc = b_dim - NB_TC
    nq = 4
    q_rows = s_dim // nq
    chunk_rows = 256
    nchunks = q_rows // chunk_rows
    ntasks1 = nb_sc * ncb * nq
    ntasks2 = nb_sc * ncb
    mesh = plsc.VectorSubcoreMesh(core_axis_name="c", subcore_axis_name="s")
    neg_inf = jnp.float32(-jnp.inf)

    @functools.partial(
        pl.kernel,
        mesh=mesh,
        out_type=jax.ShapeDtypeStruct((nb_sc, ncb, nq, TOPK, 128),
                                      jnp.float32),
        scratch_types=[
            pltpu.VMEM((chunk_rows, 128), jnp.float32),
            pltpu.VMEM((chunk_rows, 128), jnp.float32),
            pltpu.VMEM((8, TOPK, LANES), jnp.float32),
            pltpu.VMEM((TOPK, 128), jnp.float32),
            pltpu.SemaphoreType.DMA,
            pltpu.SemaphoreType.DMA,
        ],
    )
    def phase1(in_hbm, cand_hbm, slab0, slab1, tbuf, outbuf, sem0, sem1):
        wid = lax.axis_index("s") * 2 + lax.axis_index("c")
        slabs = (slab0, slab1)
        sems = (sem0, sem1)

        def chunk_src(t, ci):
            b = t // (ncb * nq)
            r = t % (ncb * nq)
            cb = r // nq
            q = r % nq
            row0 = q * q_rows + ci * chunk_rows
            return in_hbm.at[NB_TC + b, pl.ds(row0, chunk_rows),
                             pl.ds(cb * 128, 128)]

        def compute_chunk(slab, tbuf):
            def group_body(g, _):
                top = tuple(tbuf[g, k] for k in range(TOPK))

                def seg_body(i, carry):
                    rows = [slab[i * 16 + k, pl.ds(g * LANES, LANES)]
                            for k in range(16)]
                    sa = _sort_net(rows[:8], _SORT8)
                    sb = _sort_net(rows[8:], _SORT8)
                    pair = _merge_top8(sa, sb)
                    return tuple(_merge_top8(list(carry), pair))

                top = lax.fori_loop(0, chunk_rows // 16, seg_body, top)
                for k in range(TOPK):
                    tbuf[g, k] = top[k]
                return 0

            lax.fori_loop(0, 8, group_body, 0)

        @pl.when(wid < ntasks1)
        def _():
            pltpu.async_copy(chunk_src(wid, 0), slabs[0], sems[0])

        ntask_here = (ntasks1 - wid + NWORKERS - 1) // NWORKERS

        def task_body(j, _):
            t = wid + j * NWORKERS

            def init_g(g, _):
                for k in range(TOPK):
                    tbuf[g, k] = jnp.full((LANES,), neg_inf, jnp.float32)
                return 0

            lax.fori_loop(0, 8, init_g, 0)

            def pair_body(p, _):
                for sub in range(2):
                    ci = 2 * p + sub
                    ring = sub
                    nci = ci + 1
                    has_next = jnp.logical_or(
                        nci < nchunks, t + NWORKERS < ntasks1)

                    @pl.when(has_next)
                    def _():
                        nt = jnp.where(nci < nchunks, t, t + NWORKERS)
                        pltpu.async_copy(
                            chunk_src(nt, lax.rem(nci, nchunks)),
                            slabs[(ring + 1) % 2], sems[(ring + 1) % 2])

                    pltpu.make_async_copy(
                        chunk_src(t, ci), slabs[ring], sems[ring]).wait()
                    compute_chunk(slabs[ring], tbuf)
                return 0

            lax.fori_loop(0, nchunks // 2, pair_body, 0)

            b = t // (ncb * nq)
            r = t % (ncb * nq)
            cb = r // nq
            q = r % nq

            def out_g(g, _):
                for k in range(TOPK):
                    outbuf[k, pl.ds(g * LANES, LANES)] = tbuf[g, k]
                return 0

            lax.fori_loop(0, 8, out_g, 0)
            pltpu.sync_copy(outbuf, cand_hbm.at[b, cb, q])
            return 0

        lax.fori_loop(0, ntask_here, task_body, 0)

    @functools.partial(
        pl.kernel,
        mesh=mesh,
        out_type=jax.ShapeDtypeStruct((nb_sc, TOPK, c_dim), jnp.float32),
        scratch_types=[
            pltpu.VMEM((nq, TOPK, 128), jnp.float32),
            pltpu.VMEM((TOPK, 128), jnp.float32),
        ],
    )
    def phase2(cand_hbm, out_hbm, slab, outbuf):
        wid = lax.axis_index("s") * 2 + lax.axis_index("c")

        def task_body(t, _):
            b = t // ncb
            cb = t % ncb
            pltpu.sync_copy(cand_hbm.at[b, cb], slab)

            def group_body(g, _):
                qlists = [
                    [slab[q, k, pl.ds(g * LANES, LANES)]
                     for k in range(TOPK)]
                    for q in range(nq)
                ]
                m01 = _merge_top8(qlists[0], qlists[1])
                m23 = _merge_top8(qlists[2], qlists[3])
                top = _merge_top8(m01, m23)
                for k in range(TOPK):
                    outbuf[k, pl.ds(g * LANES, LANES)] = top[7 - k]
                return 0

            lax.fori_loop(0, 8, group_body, 0)
            pltpu.sync_copy(outbuf, out_hbm.at[b, :, pl.ds(cb * 128, 128)])
            return 0

        ntask_here = (ntasks2 - wid + NWORKERS - 1) // NWORKERS

        def strided_loop(i, _):
            return task_body(wid + i * NWORKERS, 0)

        lax.fori_loop(0, ntask_here, strided_loop, 0)

    tc_out = _tc_topk(inputs, NB_TC)
    cand = phase1(inputs)
    sc_out = phase2(cand)
    out_kc = jnp.concatenate([tc_out, sc_out], axis=0)
    return out_kc.transpose(0, 2, 1).reshape(b_dim, c_dim * TOPK)


# speedup vs baseline: 49.9010x; 1.0016x over previous
import functools

import jax
import jax.numpy as jnp
from jax import lax
from jax.experimental import pallas as pl
from jax.experimental.pallas import tpu as pltpu
from jax.experimental.pallas import tpu_sc as plsc

TOPK = 8
LANES = 16
NWORKERS = 32
NB_TC = 4

_SORT8 = ((0, 2), (1, 3), (4, 6), (5, 7),
          (0, 4), (1, 5), (2, 6), (3, 7),
          (0, 1), (2, 3), (4, 5), (6, 7),
          (2, 4), (3, 5), (1, 4), (3, 6),
          (1, 2), (3, 4), (5, 6))
_BITONIC8 = ((0, 4), (1, 5), (2, 6), (3, 7),
             (0, 2), (1, 3), (4, 6), (5, 7),
             (0, 1), (2, 3), (4, 5), (6, 7))


def _sort_net(vals, net):
    for i, j in net:
        a, b = vals[i], vals[j]
        vals[i] = jnp.minimum(a, b)
        vals[j] = jnp.maximum(a, b)
    return vals


def _merge_top8(top, other):
    merged = [jnp.maximum(top[m], other[7 - m]) for m in range(8)]
    return _sort_net(merged, _BITONIC8)


def _tc_block(in_ref, out_ref):
    x = in_ref[0]
    s_rows = x.shape[0]
    blk = s_rows // 8
    s = [x[blk * k:blk * (k + 1), :] for k in range(8)]
    s = _sort_net(s, _SORT8)
    length = blk
    while length > 1:
        h = length // 2
        lo = [v[:h] for v in s]
        hi = [v[h:] for v in s]
        m = [jnp.maximum(lo[i], hi[7 - i]) for i in range(8)]
        s = _sort_net(m, _BITONIC8)
        length = h
    for k in range(TOPK):
        out_ref[0, k, :] = s[7 - k][0]


def _tc_topk(inputs, nb):
    b_dim, s_dim, c_dim = inputs.shape
    return pl.pallas_call(
        _tc_block,
        grid=(nb, c_dim // 128),
        in_specs=[pl.BlockSpec((1, s_dim, 128), lambda i, j: (i, 0, j))],
        out_specs=pl.BlockSpec((1, TOPK, 128), lambda i, j: (i, 0, j)),
        out_shape=jax.ShapeDtypeStruct((nb, TOPK, c_dim), jnp.float32),
    )(inputs)


def kernel(inputs):
    b_dim, s_dim, c_dim = inputs.shape
    ncb = c_dim // 128
    nb_sc = b_dim - NB_TC
    nq = 4
    q_rows = s_dim // nq
    chunk_rows = 256
    nchunks = q_rows // chunk_rows
    ntasks1 = nb_sc * ncb * nq
    ntasks2 = nb_sc * ncb
    mesh = plsc.VectorSubcoreMesh(core_axis_name="c", subcore_axis_name="s")
    neg_inf = jnp.float32(-jnp.inf)

    @functools.partial(
        pl.kernel,
        mesh=mesh,
        out_type=jax.ShapeDtypeStruct((nb_sc, ncb, nq, TOPK, 128),
                                      jnp.float32),
        scratch_types=[
            pltpu.VMEM((chunk_rows, 128), jnp.float32),
            pltpu.VMEM((chunk_rows, 128), jnp.float32),
            pltpu.VMEM((8, TOPK, LANES), jnp.float32),
            pltpu.VMEM((TOPK, 128), jnp.float32),
            pltpu.SemaphoreType.DMA,
            pltpu.SemaphoreType.DMA,
        ],
    )
    def phase1(in_hbm, cand_hbm, slab0, slab1, tbuf, outbuf, sem0, sem1):
        wid = lax.axis_index("s") * 2 + lax.axis_index("c")
        slabs = (slab0, slab1)
        sems = (sem0, sem1)

        def chunk_src(t, ci):
            b = t // (ncb * nq)
            r = t % (ncb * nq)
            cb = r // nq
            q = r % nq
            row0 = q * q_rows + ci * chunk_rows
            return in_hbm.at[NB_TC + b, pl.ds(row0, chunk_rows),
                             pl.ds(cb * 128, 128)]

        def compute_chunk(slab, tbuf):
            def group_body(g, _):
                top = tuple(tbuf[g, k] for k in range(TOPK))

                def seg_body(i, carry):
                    rows = [slab[i * 16 + k, pl.ds(g * LANES, LANES)]
                            for k in range(16)]
                    sa = _sort_net(rows[:8], _SORT8)
                    sb = _sort_net(rows[8:], _SORT8)
                    pair = _merge_top8(sa, sb)
                    return tuple(_merge_top8(list(carry), pair))

                top = lax.fori_loop(0, chunk_rows // 16, seg_body, top)
                for k in range(TOPK):
                    tbuf[g, k] = top[k]
                return 0

            lax.fori_loop(0, 8, group_body, 0)

        @pl.when(wid < ntasks1)
        def _():
            pltpu.async_copy(chunk_src(wid, 0), slabs[0], sems[0])

        ntask_here = (ntasks1 - wid + NWORKERS - 1) // NWORKERS

        def task_body(j, _):
            t = wid + j * NWORKERS

            def init_g(g, _):
                for k in range(TOPK):
                    tbuf[g, k] = jnp.full((LANES,), neg_inf, jnp.float32)
                return 0

            lax.fori_loop(0, 8, init_g, 0)

            def pair_body(p, _):
                for sub in range(2):
                    ci = 2 * p + sub
                    ring = sub
                    nci = ci + 1
                    has_next = jnp.logical_or(
                        nci < nchunks, t + NWORKERS < ntasks1)

                    @pl.when(has_next)
                    def _():
                        nt = jnp.where(nci < nchunks, t, t + NWORKERS)
                        pltpu.async_copy(
                            chunk_src(nt, lax.rem(nci, nchunks)),
                            slabs[(ring + 1) % 2], sems[(ring + 1) % 2])

                    pltpu.make_async_copy(
                        chunk_src(t, ci), slabs[ring], sems[ring]).wait()
                    compute_chunk(slabs[ring], tbuf)
                return 0

            lax.fori_loop(0, nchunks // 2, pair_body, 0)

            b = t // (ncb * nq)
            r = t % (ncb * nq)
            cb = r // nq
            q = r % nq

            def out_g(g, _):
                for k in range(TOPK):
                    outbuf[k, pl.ds(g * LANES, LANES)] = tbuf[g, k]
                return 0

            lax.fori_loop(0, 8, out_g, 0)
            pltpu.sync_copy(outbuf, cand_hbm.at[b, cb, q])
            return 0

        lax.fori_loop(0, ntask_here, task_body, 0)

    @functools.partial(
        pl.kernel,
        mesh=mesh,
        out_type=jax.ShapeDtypeStruct((nb_sc, TOPK, c_dim), jnp.float32),
        scratch_types=[
            pltpu.VMEM((nq, TOPK, 128), jnp.float32),
            pltpu.VMEM((TOPK, 128), jnp.float32),
        ],
    )
    def phase2(cand_hbm, out_hbm, slab, outbuf):
        wid = lax.axis_index("s") * 2 + lax.axis_index("c")

        def task_body(t, _):
            b = t // ncb
            cb = t % ncb
            pltpu.sync_copy(cand_hbm.at[b, cb], slab)

            def group_body(g, _):
                qlists = [
                    [slab[q, k, pl.ds(g * LANES, LANES)]
                     for k in range(TOPK)]
                    for q in range(nq)
                ]
                m01 = _merge_top8(qlists[0], qlists[1])
                m23 = _merge_top8(qlists[2], qlists[3])
                top = _merge_top8(m01, m23)
                for k in range(TOPK):
                    outbuf[k, pl.ds(g * LANES, LANES)] = top[7 - k]
                return 0

            lax.fori_loop(0, 8, group_body, 0)
            pltpu.sync_copy(outbuf, out_hbm.at[b, :, pl.ds(cb * 128, 128)])
            return 0

        ntask_here = (ntasks2 - wid + NWORKERS - 1) // NWORKERS

        def strided_loop(i, _):
            return task_body(wid + i * NWORKERS, 0)

        lax.fori_loop(0, ntask_here, strided_loop, 0)

    cand = phase1(inputs)
    tc_out = _tc_topk(inputs, NB_TC)
    sc_out = phase2(cand)
    out_kc = jnp.concatenate([tc_out, sc_out], axis=0)
    return out_kc.transpose(0, 2, 1).reshape(b_dim, c_dim * TOPK)


# speedup vs baseline: 72.2531x; 1.4479x over previous
import functools

import jax
import jax.numpy as jnp
from jax import lax
from jax.experimental import pallas as pl
from jax.experimental.pallas import tpu as pltpu
from jax.experimental.pallas import tpu_sc as plsc

TOPK = 8
LANES = 16
NSUB = 16
NB_TC = 4

_SORT8 = ((0, 2), (1, 3), (4, 6), (5, 7),
          (0, 4), (1, 5), (2, 6), (3, 7),
          (0, 1), (2, 3), (4, 5), (6, 7),
          (2, 4), (3, 5), (1, 4), (3, 6),
          (1, 2), (3, 4), (5, 6))
_BITONIC8 = ((0, 4), (1, 5), (2, 6), (3, 7),
             (0, 2), (1, 3), (4, 6), (5, 7),
             (0, 1), (2, 3), (4, 5), (6, 7))


def _sort_net(vals, net):
    for i, j in net:
        a, b = vals[i], vals[j]
        vals[i] = jnp.minimum(a, b)
        vals[j] = jnp.maximum(a, b)
    return vals


def _merge_top8(top, other):
    merged = [jnp.maximum(top[m], other[7 - m]) for m in range(8)]
    return _sort_net(merged, _BITONIC8)


def _tc_block(in_ref, out_ref):
    x = in_ref[0]
    s_rows = x.shape[0]
    blk = s_rows // 8
    s = [x[blk * k:blk * (k + 1), :] for k in range(8)]
    s = _sort_net(s, _SORT8)
    length = blk
    while length > 1:
        h = length // 2
        lo = [v[:h] for v in s]
        hi = [v[h:] for v in s]
        m = [jnp.maximum(lo[i], hi[7 - i]) for i in range(8)]
        s = _sort_net(m, _BITONIC8)
        length = h
    for k in range(TOPK):
        out_ref[0, k, :] = s[7 - k][0]


def _tc_topk(inputs, nb):
    b_dim, s_dim, c_dim = inputs.shape
    return pl.pallas_call(
        _tc_block,
        grid=(nb, c_dim // 128),
        in_specs=[pl.BlockSpec((1, s_dim, 128), lambda i, j: (i, 0, j))],
        out_specs=pl.BlockSpec((1, TOPK, 128), lambda i, j: (i, 0, j)),
        out_shape=jax.ShapeDtypeStruct((nb, TOPK, c_dim), jnp.float32),
    )(inputs)


def kernel(inputs):
    b_dim, s_dim, c_dim = inputs.shape
    ncb = c_dim // 128
    nb_sc = b_dim - NB_TC
    nq = 4
    q_rows = s_dim // nq
    chunk_rows = 256
    nchunks = q_rows // chunk_rows
    npairs = nb_sc * ncb
    pairs_per_sc = npairs // 2
    nq_sc = pairs_per_sc * nq
    mesh = plsc.VectorSubcoreMesh(core_axis_name="c", subcore_axis_name="s")
    neg_inf = jnp.float32(-jnp.inf)

    @functools.partial(
        pl.kernel,
        mesh=mesh,
        out_type=(
            jax.ShapeDtypeStruct((nb_sc, ncb, nq, TOPK, 128), jnp.float32),
            jax.ShapeDtypeStruct((nb_sc, TOPK, c_dim), jnp.float32),
        ),
        scratch_types=[
            pltpu.VMEM((chunk_rows, 128), jnp.float32),
            pltpu.VMEM((chunk_rows, 128), jnp.float32),
            pltpu.VMEM((8, TOPK, LANES), jnp.float32),
            pltpu.VMEM((TOPK, 128), jnp.float32),
            pltpu.VMEM((nq, TOPK, 128), jnp.float32),
            pltpu.SemaphoreType.DMA,
            pltpu.SemaphoreType.DMA,
        ],
    )
    def sc_topk(in_hbm, cand_hbm, out_hbm, slab0, slab1, tbuf, outbuf,
                mslab, sem0, sem1):
        cid = lax.axis_index("c")
        sid = lax.axis_index("s")
        slabs = (slab0, slab1)
        sems = (sem0, sem1)

        def task_coords(t_loc):
            pg = cid * pairs_per_sc + t_loc // nq
            q = t_loc % nq
            return pg // ncb, pg % ncb, q

        def chunk_src(t_loc, ci):
            b, cb, q = task_coords(t_loc)
            row0 = q * q_rows + ci * chunk_rows
            return in_hbm.at[NB_TC + b, pl.ds(row0, chunk_rows),
                             pl.ds(cb * 128, 128)]

        def compute_chunk(slab, tbuf):
            def group_body(g, _):
                top = tuple(tbuf[g, k] for k in range(TOPK))

                def seg_body(i, carry):
                    rows = [slab[i * 16 + k, pl.ds(g * LANES, LANES)]
                            for k in range(16)]
                    sa = _sort_net(rows[:8], _SORT8)
                    sb = _sort_net(rows[8:], _SORT8)
                    pair = _merge_top8(sa, sb)
                    return tuple(_merge_top8(list(carry), pair))

                top = lax.fori_loop(0, chunk_rows // 16, seg_body, top)
                for k in range(TOPK):
                    tbuf[g, k] = top[k]
                return 0

            lax.fori_loop(0, 8, group_body, 0)

        @pl.when(sid < nq_sc)
        def _():
            pltpu.async_copy(chunk_src(sid, 0), slabs[0], sems[0])

        ntask_here = (nq_sc - sid + NSUB - 1) // NSUB

        def task_body(j, _):
            t = sid + j * NSUB

            def init_g(g, _):
                for k in range(TOPK):
                    tbuf[g, k] = jnp.full((LANES,), neg_inf, jnp.float32)
                return 0

            lax.fori_loop(0, 8, init_g, 0)

            def pair_body(p, _):
                for sub in range(2):
                    ci = 2 * p + sub
                    ring = sub
                    nci = ci + 1
                    has_next = jnp.logical_or(
                        nci < nchunks, t + NSUB < nq_sc)

                    @pl.when(has_next)
                    def _():
                        nt = jnp.where(nci < nchunks, t, t + NSUB)
                        pltpu.async_copy(
                            chunk_src(nt, lax.rem(nci, nchunks)),
                            slabs[(ring + 1) % 2], sems[(ring + 1) % 2])

                    pltpu.make_async_copy(
                        chunk_src(t, ci), slabs[ring], sems[ring]).wait()
                    compute_chunk(slabs[ring], tbuf)
                return 0

            lax.fori_loop(0, nchunks // 2, pair_body, 0)

            b, cb, q = task_coords(t)

            def out_g(g, _):
                for k in range(TOPK):
                    outbuf[k, pl.ds(g * LANES, LANES)] = tbuf[g, k]
                return 0

            lax.fori_loop(0, 8, out_g, 0)
            pltpu.sync_copy(outbuf, cand_hbm.at[b, cb, q])
            return 0

        lax.fori_loop(0, ntask_here, task_body, 0)

        plsc.subcore_barrier()

        nmerge_here = (pairs_per_sc - sid + NSUB - 1) // NSUB

        def merge_body(j, _):
            pg = cid * pairs_per_sc + sid + j * NSUB
            b = pg // ncb
            cb = pg % ncb
            pltpu.sync_copy(cand_hbm.at[b, cb], mslab)

            def group_body(g, _):
                qlists = [
                    [mslab[q, k, pl.ds(g * LANES, LANES)]
                     for k in range(TOPK)]
                    for q in range(nq)
                ]
                m01 = _merge_top8(qlists[0], qlists[1])
                m23 = _merge_top8(qlists[2], qlists[3])
                top = _merge_top8(m01, m23)
                for k in range(TOPK):
                    outbuf[k, pl.ds(g * LANES, LANES)] = top[7 - k]
                return 0

            lax.fori_loop(0, 8, group_body, 0)
            pltpu.sync_copy(outbuf, out_hbm.at[b, :, pl.ds(cb * 128, 128)])
            return 0

        lax.fori_loop(0, nmerge_here, merge_body, 0)

    _, sc_out = sc_topk(inputs)
    tc_out = _tc_topk(inputs, NB_TC)
    out_kc = jnp.concatenate([tc_out, sc_out], axis=0)
    return out_kc.transpose(0, 2, 1).reshape(b_dim, c_dim * TOPK)


# speedup vs baseline: 77.6747x; 1.0750x over previous
import functools

import jax
import jax.numpy as jnp
from jax import lax
from jax.experimental import pallas as pl
from jax.experimental.pallas import tpu as pltpu
from jax.experimental.pallas import tpu_sc as plsc

TOPK = 8
LANES = 16
NSUB = 16
P_TC = 28

_SORT8 = ((0, 2), (1, 3), (4, 6), (5, 7),
          (0, 4), (1, 5), (2, 6), (3, 7),
          (0, 1), (2, 3), (4, 5), (6, 7),
          (2, 4), (3, 5), (1, 4), (3, 6),
          (1, 2), (3, 4), (5, 6))
_BITONIC8 = ((0, 4), (1, 5), (2, 6), (3, 7),
             (0, 2), (1, 3), (4, 6), (5, 7),
             (0, 1), (2, 3), (4, 5), (6, 7))


def _sort_net(vals, net):
    for i, j in net:
        a, b = vals[i], vals[j]
        vals[i] = jnp.minimum(a, b)
        vals[j] = jnp.maximum(a, b)
    return vals


def _merge_top8(top, other):
    merged = [jnp.maximum(top[m], other[7 - m]) for m in range(8)]
    return _sort_net(merged, _BITONIC8)


def _tc_block(in_ref, out_ref):
    x = in_ref[0]
    s_rows = x.shape[0]
    blk = s_rows // 8
    s = [x[blk * k:blk * (k + 1), :] for k in range(8)]
    s = _sort_net(s, _SORT8)
    length = blk
    while length > 1:
        h = length // 2
        lo = [v[:h] for v in s]
        hi = [v[h:] for v in s]
        m = [jnp.maximum(lo[i], hi[7 - i]) for i in range(8)]
        s = _sort_net(m, _BITONIC8)
        length = h
    for k in range(TOPK):
        out_ref[0, k, :] = s[7 - k][0]


def _tc_topk(inputs, npair, ncb):
    b_dim, s_dim, c_dim = inputs.shape
    return pl.pallas_call(
        _tc_block,
        grid=(npair,),
        in_specs=[pl.BlockSpec((1, s_dim, 128),
                               lambda i: (i // ncb, 0, i % ncb))],
        out_specs=pl.BlockSpec((1, TOPK, 128), lambda i: (i, 0, 0)),
        out_shape=jax.ShapeDtypeStruct((npair, TOPK, 128), jnp.float32),
    )(inputs)


def kernel(inputs):
    b_dim, s_dim, c_dim = inputs.shape
    ncb = c_dim // 128
    npairs_all = b_dim * ncb
    np_sc = npairs_all - P_TC
    pairs_per_sc = np_sc // 2
    nq = 8
    q_rows = s_dim // nq
    chunk_rows = 256
    nchunks = q_rows // chunk_rows
    nq_sc = pairs_per_sc * nq
    mesh = plsc.VectorSubcoreMesh(core_axis_name="c", subcore_axis_name="s")
    neg_inf = jnp.float32(-jnp.inf)

    @functools.partial(
        pl.kernel,
        mesh=mesh,
        out_type=(
            jax.ShapeDtypeStruct((np_sc, nq, TOPK, 128), jnp.float32),
            jax.ShapeDtypeStruct((np_sc, TOPK, 128), jnp.float32),
        ),
        scratch_types=[
            pltpu.VMEM((chunk_rows, 128), jnp.float32),
            pltpu.VMEM((chunk_rows, 128), jnp.float32),
            pltpu.VMEM((8, TOPK, LANES), jnp.float32),
            pltpu.VMEM((TOPK, 128), jnp.float32),
            pltpu.VMEM((nq, TOPK, 128), jnp.float32),
            pltpu.SemaphoreType.DMA,
            pltpu.SemaphoreType.DMA,
        ],
    )
    def sc_topk(in_hbm, cand_hbm, out_hbm, slab0, slab1, tbuf, outbuf,
                mslab, sem0, sem1):
        cid = lax.axis_index("c")
        sid = lax.axis_index("s")
        slabs = (slab0, slab1)
        sems = (sem0, sem1)

        def task_coords(t_loc):
            pl_idx = cid * pairs_per_sc + t_loc // nq
            pg = P_TC + pl_idx
            return pl_idx, pg // ncb, pg % ncb, t_loc % nq

        def chunk_src(t_loc, ci):
            _, b, cb, q = task_coords(t_loc)
            row0 = q * q_rows + ci * chunk_rows
            return in_hbm.at[b, pl.ds(row0, chunk_rows),
                             pl.ds(cb * 128, 128)]

        def compute_chunk(slab, tbuf):
            def group_body(g, _):
                top = tuple(tbuf[g, k] for k in range(TOPK))

                def seg_body(i, carry):
                    rows = [slab[i * 16 + k, pl.ds(g * LANES, LANES)]
                            for k in range(16)]
                    sa = _sort_net(rows[:8], _SORT8)
                    sb = _sort_net(rows[8:], _SORT8)
                    pair = _merge_top8(sa, sb)
                    return tuple(_merge_top8(list(carry), pair))

                top = lax.fori_loop(0, chunk_rows // 16, seg_body, top)
                for k in range(TOPK):
                    tbuf[g, k] = top[k]
                return 0

            lax.fori_loop(0, 8, group_body, 0)

        @pl.when(sid < nq_sc)
        def _():
            pltpu.async_copy(chunk_src(sid, 0), slabs[0], sems[0])

        ntask_here = (nq_sc - sid + NSUB - 1) // NSUB

        def task_body(j, _):
            t = sid + j * NSUB

            def init_g(g, _):
                for k in range(TOPK):
                    tbuf[g, k] = jnp.full((LANES,), neg_inf, jnp.float32)
                return 0

            lax.fori_loop(0, 8, init_g, 0)

            def pair_body(p, _):
                for sub in range(2):
                    ci = 2 * p + sub
                    ring = sub
                    nci = ci + 1
                    has_next = jnp.logical_or(
                        nci < nchunks, t + NSUB < nq_sc)

                    @pl.when(has_next)
                    def _():
                        nt = jnp.where(nci < nchunks, t, t + NSUB)
                        pltpu.async_copy(
                            chunk_src(nt, lax.rem(nci, nchunks)),
                            slabs[(ring + 1) % 2], sems[(ring + 1) % 2])

                    pltpu.make_async_copy(
                        chunk_src(t, ci), slabs[ring], sems[ring]).wait()
                    compute_chunk(slabs[ring], tbuf)
                return 0

            lax.fori_loop(0, nchunks // 2, pair_body, 0)

            pl_idx, _, _, q = task_coords(t)

            def out_g(g, _):
                for k in range(TOPK):
                    outbuf[k, pl.ds(g * LANES, LANES)] = tbuf[g, k]
                return 0

            lax.fori_loop(0, 8, out_g, 0)
            pltpu.sync_copy(outbuf, cand_hbm.at[pl_idx, q])
            return 0

        lax.fori_loop(0, ntask_here, task_body, 0)

        plsc.subcore_barrier()

        nmerge_here = (pairs_per_sc - sid + NSUB - 1) // NSUB

        def merge_body(j, _):
            pl_idx = cid * pairs_per_sc + sid + j * NSUB
            pltpu.sync_copy(cand_hbm.at[pl_idx], mslab)

            def group_body(g, _):
                qlists = [
                    [mslab[q, k, pl.ds(g * LANES, LANES)]
                     for k in range(TOPK)]
                    for q in range(nq)
                ]
                while len(qlists) > 1:
                    qlists = [_merge_top8(qlists[2 * i], qlists[2 * i + 1])
                              for i in range(len(qlists) // 2)]
                top = qlists[0]
                for k in range(TOPK):
                    outbuf[k, pl.ds(g * LANES, LANES)] = top[7 - k]
                return 0

            lax.fori_loop(0, 8, group_body, 0)
            pltpu.sync_copy(outbuf, out_hbm.at[pl_idx])
            return 0

        lax.fori_loop(0, nmerge_here, merge_body, 0)

    _, sc_out = sc_topk(inputs)
    tc_out = _tc_topk(inputs, P_TC, ncb)
    allp = jnp.concatenate([tc_out, sc_out], axis=0)
    out = allp.reshape(b_dim, ncb, TOPK, 128).transpose(0, 1, 3, 2)
    return out.reshape(b_dim, c_dim * TOPK)
